# Initial kernel scaffold; baseline (speedup 1.0000x reference)
import jax
import jax.numpy as jnp
from jax.experimental import pallas as pl


def kernel(x, edge_index, pitch_speed, Wgp, bgp, Wg_f, bg_f, Wb_f, bb_f, Wg_d, bg_d, Wb_d, bb_d, Wfe_s, Wfe_n, bfe, Wds_s, Wds_n, bds, Wis_s, Wis_n, bis, num_timesteps):
    raise NotImplementedError("write your pallas kernel here")


# speedup vs baseline: 8.3468x; 8.3468x over previous
import functools

import jax
import jax.numpy as jnp
from jax import lax
from jax.experimental import pallas as pl
from jax.experimental.pallas import tpu as pltpu
from jax.experimental.pallas import tpu_sc as plsc

N = 100000
E = 1600000
D = 9
H = 128
G = 64
W16 = 16

NC = 2
NS = 16
NW = NC * NS
EPW = E // NW
CH = 128
FULL_ITERS = EPW // CH
TAIL = EPW - FULL_ITERS * CH
NP = 100096
RPT = NP // NS

BN = 2000
GRID = N // BN


@functools.cache
def _build_sc_scatter():
    mesh = plsc.VectorSubcoreMesh(core_axis_name="c", subcore_axis_name="s")

    @functools.partial(
        pl.kernel,
        out_type=jax.ShapeDtypeStruct((NC, NP, W16), jnp.float32),
        mesh=mesh,
        compiler_params=pltpu.CompilerParams(use_tc_tiling_on_sc=False),
        scratch_types=[
            pltpu.VMEM((CH,), jnp.int32),
            pltpu.VMEM((CH,), jnp.int32),
            pltpu.VMEM((CH, W16), jnp.float32),
            pltpu.VMEM((TAIL,), jnp.int32),
            pltpu.VMEM((TAIL,), jnp.int32),
            pltpu.VMEM((TAIL, W16), jnp.float32),
            pltpu.VMEM_SHARED((NP, W16), jnp.float32),
            pltpu.SemaphoreType.DMA,
        ],
    )
    def sc_scatter(payload_hbm, src_hbm, dst_hbm, zeros_hbm, out_hbm,
                   src_v, dst_v, rows_v, src_t, dst_t, rows_t, acc, sem):
        c = lax.axis_index("c")
        s = lax.axis_index("s")
        w = c * NS + s
        base = w * EPW
        row0 = s * RPT

        pltpu.sync_copy(zeros_hbm, acc.at[pl.ds(row0, RPT)])
        plsc.subcore_barrier()

        def body(i, carry):
            off = base + i * CH
            pltpu.sync_copy(src_hbm.at[pl.ds(off, CH)], src_v)
            pltpu.sync_copy(dst_hbm.at[pl.ds(off, CH)], dst_v)
            pltpu.async_copy(payload_hbm.at[src_v], rows_v, sem).wait()
            pltpu.sync_copy(rows_v, acc.at[dst_v], add=True)
            return carry

        lax.fori_loop(0, FULL_ITERS, body, 0)

        offt = base + FULL_ITERS * CH
        pltpu.sync_copy(src_hbm.at[pl.ds(offt, TAIL)], src_t)
        pltpu.sync_copy(dst_hbm.at[pl.ds(offt, TAIL)], dst_t)
        pltpu.async_copy(payload_hbm.at[src_t], rows_t, sem).wait()
        pltpu.sync_copy(rows_t, acc.at[dst_t], add=True)

        plsc.subcore_barrier()
        pltpu.sync_copy(acc.at[pl.ds(row0, RPT)],
                        out_hbm.at[c, pl.ds(row0, RPT)])

    return sc_scatter


def _sc_scatter(payload, src, dst, zeros_rt):
    return _build_sc_scatter()(payload, src, dst, zeros_rt)


def _mean_from_partials(agg_ref):
    aggs = agg_ref[0] + agg_ref[1]
    deg = aggs[:, 9:10]
    inv = 1.0 / jnp.maximum(deg, 1.0)
    return aggs * inv


def _stage_a_body(cur_ref, agg_ref, wfes_ref, wfen_ref, bfe_ref,
                  gf_ref, bf_ref, gd_ref, bd_ref, m9_ref,
                  an_ref, as_ref, bn_ref, bs_ref, c2n_ref, c2s_ref,
                  p2_ref, s2_ref, fpu_ref):
    cur = cur_ref[...]
    mean1 = _mean_from_partials(agg_ref)
    pre = (jnp.dot(cur, wfes_ref[...], preferred_element_type=jnp.float32)
           + jnp.dot(mean1, wfen_ref[...], preferred_element_type=jnp.float32)
           + bfe_ref[...])
    learned = jnp.maximum(pre, 0.0)
    mu = jnp.mean(learned, axis=1, keepdims=True)
    var = jnp.mean((learned - mu) ** 2, axis=1, keepdims=True)
    ln = (learned - mu) * lax.rsqrt(var + 1e-5) * gf_ref[...] + bf_ref[...]

    m9 = m9_ref[...]
    mu9 = jnp.sum(cur * m9, axis=1, keepdims=True) / 9.0
    var9 = jnp.sum((cur - mu9) ** 2 * m9, axis=1, keepdims=True) / 9.0
    fpu = ((cur - mu9) * lax.rsqrt(var9 + 1e-5) * gd_ref[...] + bd_ref[...]) * m9

    p2_ref[...] = (jnp.dot(ln, an_ref[...], preferred_element_type=jnp.float32)
                   + jnp.dot(fpu, bn_ref[...], preferred_element_type=jnp.float32)
                   + c2n_ref[...])
    s2_ref[...] = (jnp.dot(ln, as_ref[...], preferred_element_type=jnp.float32)
                   + jnp.dot(fpu, bs_ref[...], preferred_element_type=jnp.float32)
                   + c2s_ref[...])
    fpu_ref[...] = fpu


def _stage_b_body(agg_ref, s2_ref, wisn_ref, wiss_ref, bis_ref, e9_ref,
                  p3_ref, s3_ref):
    fdot = s2_ref[...] + _mean_from_partials(agg_ref)
    p3_ref[...] = jnp.dot(fdot, wisn_ref[...],
                          preferred_element_type=jnp.float32) + e9_ref[...]
    s3_ref[...] = jnp.dot(fdot, wiss_ref[...],
                          preferred_element_type=jnp.float32) + bis_ref[...]


def _stage_c_body(agg_ref, s3_ref, fpu_ref, m9_ref, e9_ref, pred_ref, cur_ref):
    fint = s3_ref[...] + _mean_from_partials(agg_ref)
    fpred = (fpu_ref[...] + fint) * m9_ref[...]
    pred_ref[...] = fpred[:, :D]
    cur_ref[...] = fpred + e9_ref[...]


def _row_spec(width):
    return pl.BlockSpec((BN, width), lambda i: (i, 0))


def _agg_spec():
    return pl.BlockSpec((NC, BN, W16), lambda i: (0, i, 0))


def _full_spec(shape):
    nd = len(shape)
    return pl.BlockSpec(shape, lambda i, _n=nd: (0,) * _n)


def _stage_a(cur, agg, wfes, wfen, bfe, gf, bf, gd, bd, m9,
             an, as_, bn, bs, c2n, c2s):
    return pl.pallas_call(
        _stage_a_body,
        grid=(GRID,),
        in_specs=[
            _row_spec(W16), _agg_spec(),
            _full_spec((W16, H)), _full_spec((W16, H)), _full_spec((1, H)),
            _full_spec((1, H)), _full_spec((1, H)),
            _full_spec((1, W16)), _full_spec((1, W16)), _full_spec((1, W16)),
            _full_spec((H, W16)), _full_spec((H, W16)),
            _full_spec((W16, W16)), _full_spec((W16, W16)),
            _full_spec((1, W16)), _full_spec((1, W16)),
        ],
        out_specs=[_row_spec(W16), _row_spec(W16), _row_spec(W16)],
        out_shape=[
            jax.ShapeDtypeStruct((N, W16), jnp.float32),
            jax.ShapeDtypeStruct((N, W16), jnp.float32),
            jax.ShapeDtypeStruct((N, W16), jnp.float32),
        ],
    )(cur, agg, wfes, wfen, bfe, gf, bf, gd, bd, m9, an, as_, bn, bs, c2n, c2s)


def _stage_b(agg, s2, wisn, wiss, bis16, e9):
    return pl.pallas_call(
        _stage_b_body,
        grid=(GRID,),
        in_specs=[
            _agg_spec(), _row_spec(W16),
            _full_spec((W16, W16)), _full_spec((W16, W16)),
            _full_spec((1, W16)), _full_spec((1, W16)),
        ],
        out_specs=[_row_spec(W16), _row_spec(W16)],
        out_shape=[
            jax.ShapeDtypeStruct((N, W16), jnp.float32),
            jax.ShapeDtypeStruct((N, W16), jnp.float32),
        ],
    )(agg, s2, wisn, wiss, bis16, e9)


def _stage_c(agg, s3, fpu, m9, e9):
    return pl.pallas_call(
        _stage_c_body,
        grid=(GRID,),
        in_specs=[
            _agg_spec(), _row_spec(W16), _row_spec(W16), _full_spec((1, W16)),
            _full_spec((1, W16)),
        ],
        out_specs=[_row_spec(D), _row_spec(W16)],
        out_shape=[
            jax.ShapeDtypeStruct((N, D), jnp.float32),
            jax.ShapeDtypeStruct((N, W16), jnp.float32),
        ],
    )(agg, s3, fpu, m9, e9)


def _pad16_vec(v):
    return jnp.pad(v, (0, W16 - v.shape[0])).reshape(1, W16)


def kernel(x, edge_index, pitch_speed, Wgp, bgp, Wg_f, bg_f, Wb_f, bb_f,
           Wg_d, bg_d, Wb_d, bb_d, Wfe_s, Wfe_n, bfe, Wds_s, Wds_n, bds,
           Wis_s, Wis_n, bis, num_timesteps):
    del num_timesteps
    src = edge_index[0]
    dst = edge_index[1]

    ge = jnp.tanh(pitch_speed @ Wgp + bgp)
    gf = (pitch_speed @ Wg_f + bg_f).reshape(1, H)
    bf = (pitch_speed @ Wb_f + bb_f).reshape(1, H)
    gd = _pad16_vec(pitch_speed @ Wg_d + bg_d)
    bd = _pad16_vec(pitch_speed @ Wb_d + bb_d)

    wfes = jnp.pad(Wfe_s, ((0, W16 - D), (0, 0)))
    wfen = jnp.pad(Wfe_n, ((0, W16 - D), (0, 0)))
    bfe_r = bfe.reshape(1, H)

    as_ = jnp.pad(Wds_s[:H], ((0, 0), (0, W16 - D)))
    an = jnp.pad(Wds_n[:H], ((0, 0), (0, W16 - D)))
    bs = jnp.pad(Wds_s[H:H + D], ((0, W16 - D), (0, W16 - D)))
    bn = jnp.pad(Wds_n[H:H + D], ((0, W16 - D), (0, W16 - D)))
    e9 = jnp.zeros((1, W16), jnp.float32).at[0, D].set(1.0)
    c2s = _pad16_vec(ge @ Wds_s[H + D:] + bds)
    c2n = _pad16_vec(ge @ Wds_n[H + D:]) + e9

    wiss = jnp.pad(Wis_s, ((0, W16 - D), (0, W16 - D)))
    wisn = jnp.pad(Wis_n, ((0, W16 - D), (0, W16 - D)))
    bis16 = _pad16_vec(bis)

    m9 = jnp.concatenate(
        [jnp.ones((1, D), jnp.float32),
         jnp.zeros((1, W16 - D), jnp.float32)], axis=1)

    zeros_rt = jnp.zeros((RPT, W16), jnp.float32)
    cur = jnp.concatenate(
        [x[:, :D], jnp.ones((N, 1), jnp.float32),
         jnp.zeros((N, W16 - D - 1), jnp.float32)], axis=1)

    preds = []
    for _t in range(4):
        agg1 = _sc_scatter(cur, src, dst, zeros_rt)
        p2, s2, fpu = _stage_a(cur, agg1, wfes, wfen, bfe_r, gf, bf, gd, bd,
                               m9, an, as_, bn, bs, c2n, c2s)
        agg2 = _sc_scatter(p2, src, dst, zeros_rt)
        p3, s3 = _stage_b(agg2, s2, wisn, wiss, bis16, e9)
        agg3 = _sc_scatter(p3, src, dst, zeros_rt)
        pred, cur = _stage_c(agg3, s3, fpu, m9, e9)
        preds.append(pred)

    return jnp.stack(preds)


# speedup vs baseline: 25.4781x; 3.0524x over previous
import functools

import jax
import jax.numpy as jnp
from jax import lax
from jax.experimental import pallas as pl
from jax.experimental.pallas import tpu as pltpu
from jax.experimental.pallas import tpu_sc as plsc

N = 100000
E = 1600000
D = 9
H = 128
G = 64
W16 = 16

NC = 2
NS = 16
NW = NC * NS
CH = 128
GB = 28
GPW = 14
CPW = GB * GPW
EP = NW * CPW * CH
NB = 7
NP = 100096
RPT = NP // NS

BN = 2000
GRID = N // BN


@functools.cache
def _build_sc_scatter():
    mesh = plsc.VectorSubcoreMesh(core_axis_name="c", subcore_axis_name="s")

    @functools.partial(
        pl.kernel,
        out_type=jax.ShapeDtypeStruct((NC, NP, W16), jnp.float32),
        mesh=mesh,
        compiler_params=pltpu.CompilerParams(use_tc_tiling_on_sc=False),
        scratch_types=[
            pltpu.VMEM((2, GB, CH), jnp.int32),
            pltpu.VMEM((2, GB, CH), jnp.int32),
            pltpu.VMEM((NB, CH, W16), jnp.float32),
            pltpu.VMEM_SHARED((NP, W16), jnp.float32),
            pltpu.SemaphoreType.DMA((2,)),
            pltpu.SemaphoreType.DMA((NB,)),
            pltpu.SemaphoreType.DMA((NB,)),
        ],
    )
    def sc_scatter(payload_hbm, src_hbm, dst_hbm, zeros_hbm, out_hbm,
                   sidx, didx, rows, acc, sem_i, sem_g, sem_s):
        c = lax.axis_index("c")
        s = lax.axis_index("s")
        w = c * NS + s
        row0 = s * RPT
        g0 = w * GPW

        def fire_group_fetch(g, gb):
            r = (g0 + g) * GB
            pltpu.async_copy(src_hbm.at[pl.ds(r, GB)], sidx.at[gb],
                             sem_i.at[gb])
            pltpu.async_copy(dst_hbm.at[pl.ds(r, GB)], didx.at[gb],
                             sem_i.at[gb])

        def wait_group_fetch(g, gb):
            r = (g0 + g) * GB
            pltpu.make_async_copy(src_hbm.at[pl.ds(r, GB)], sidx.at[gb],
                                  sem_i.at[gb]).wait()
            pltpu.make_async_copy(dst_hbm.at[pl.ds(r, GB)], didx.at[gb],
                                  sem_i.at[gb]).wait()

        fire_group_fetch(0, 0)
        pltpu.sync_copy(zeros_hbm, acc.at[pl.ds(row0, RPT)])
        plsc.subcore_barrier()

        def group_body(g, carry):
            gb = lax.rem(g, 2)
            wait_group_fetch(g, gb)

            @pl.when(g + 1 < GPW)
            def _():
                fire_group_fetch(g + 1, lax.rem(g + 1, 2))

            for b in range(NB):
                pltpu.async_copy(payload_hbm.at[sidx.at[gb].at[b]],
                                 rows.at[b], sem_g.at[b])

            def chunk_body(j, carry2):
                b = lax.rem(j, NB)
                pltpu.make_async_copy(payload_hbm.at[sidx.at[gb].at[j]],
                                      rows.at[b], sem_g.at[b]).wait()
                pltpu.async_copy(rows.at[b], acc.at[didx.at[gb].at[j]],
                                 sem_s.at[b], add=True)
                pltpu.make_async_copy(rows.at[b], acc.at[didx.at[gb].at[j]],
                                      sem_s.at[b]).wait()

                @pl.when(j + NB < GB)
                def _():
                    pltpu.async_copy(payload_hbm.at[sidx.at[gb].at[j + NB]],
                                     rows.at[b], sem_g.at[b])

                return carry2

            lax.fori_loop(0, GB, chunk_body, 0)
            return carry

        lax.fori_loop(0, GPW, group_body, 0)

        plsc.subcore_barrier()
        pltpu.sync_copy(acc.at[pl.ds(row0, RPT)],
                        out_hbm.at[c, pl.ds(row0, RPT)])

    return sc_scatter


def _sc_scatter(payload, src, dst, zeros_rt):
    return _build_sc_scatter()(payload, src, dst, zeros_rt)


def _mean_from_partials(agg_ref):
    aggs = agg_ref[0] + agg_ref[1]
    deg = aggs[:, 9:10]
    inv = 1.0 / jnp.maximum(deg, 1.0)
    return aggs * inv


def _stage_a_body(cur_ref, agg_ref, wfes_ref, wfen_ref, bfe_ref,
                  gf_ref, bf_ref, gd_ref, bd_ref, m9_ref,
                  an_ref, as_ref, bn_ref, bs_ref, c2n_ref, c2s_ref,
                  p2_ref, s2_ref, fpu_ref):
    cur = cur_ref[...]
    mean1 = _mean_from_partials(agg_ref)
    pre = (jnp.dot(cur, wfes_ref[...], preferred_element_type=jnp.float32)
           + jnp.dot(mean1, wfen_ref[...], preferred_element_type=jnp.float32)
           + bfe_ref[...])
    learned = jnp.maximum(pre, 0.0)
    mu = jnp.mean(learned, axis=1, keepdims=True)
    var = jnp.mean((learned - mu) ** 2, axis=1, keepdims=True)
    ln = (learned - mu) * lax.rsqrt(var + 1e-5) * gf_ref[...] + bf_ref[...]

    m9 = m9_ref[...]
    mu9 = jnp.sum(cur * m9, axis=1, keepdims=True) / 9.0
    var9 = jnp.sum((cur - mu9) ** 2 * m9, axis=1, keepdims=True) / 9.0
    fpu = ((cur - mu9) * lax.rsqrt(var9 + 1e-5) * gd_ref[...] + bd_ref[...]) * m9

    p2_ref[...] = (jnp.dot(ln, an_ref[...], preferred_element_type=jnp.float32)
                   + jnp.dot(fpu, bn_ref[...], preferred_element_type=jnp.float32)
                   + c2n_ref[...])
    s2_ref[...] = (jnp.dot(ln, as_ref[...], preferred_element_type=jnp.float32)
                   + jnp.dot(fpu, bs_ref[...], preferred_element_type=jnp.float32)
                   + c2s_ref[...])
    fpu_ref[...] = fpu


def _stage_b_body(agg_ref, s2_ref, wisn_ref, wiss_ref, bis_ref, e9_ref,
                  p3_ref, s3_ref):
    fdot = s2_ref[...] + _mean_from_partials(agg_ref)
    p3_ref[...] = jnp.dot(fdot, wisn_ref[...],
                          preferred_element_type=jnp.float32) + e9_ref[...]
    s3_ref[...] = jnp.dot(fdot, wiss_ref[...],
                          preferred_element_type=jnp.float32) + bis_ref[...]


def _stage_c_body(agg_ref, s3_ref, fpu_ref, m9_ref, e9_ref, pred_ref, cur_ref):
    fint = s3_ref[...] + _mean_from_partials(agg_ref)
    fpred = (fpu_ref[...] + fint) * m9_ref[...]
    pred_ref[...] = fpred[:, :D]
    cur_ref[...] = fpred + e9_ref[...]


def _row_spec(width):
    return pl.BlockSpec((BN, width), lambda i: (i, 0))


def _agg_spec():
    return pl.BlockSpec((NC, BN, W16), lambda i: (0, i, 0))


def _full_spec(shape):
    nd = len(shape)
    return pl.BlockSpec(shape, lambda i, _n=nd: (0,) * _n)


def _stage_a(cur, agg, wfes, wfen, bfe, gf, bf, gd, bd, m9,
             an, as_, bn, bs, c2n, c2s):
    return pl.pallas_call(
        _stage_a_body,
        grid=(GRID,),
        in_specs=[
            _row_spec(W16), _agg_spec(),
            _full_spec((W16, H)), _full_spec((W16, H)), _full_spec((1, H)),
            _full_spec((1, H)), _full_spec((1, H)),
            _full_spec((1, W16)), _full_spec((1, W16)), _full_spec((1, W16)),
            _full_spec((H, W16)), _full_spec((H, W16)),
            _full_spec((W16, W16)), _full_spec((W16, W16)),
            _full_spec((1, W16)), _full_spec((1, W16)),
        ],
        out_specs=[_row_spec(W16), _row_spec(W16), _row_spec(W16)],
        out_shape=[
            jax.ShapeDtypeStruct((N, W16), jnp.float32),
            jax.ShapeDtypeStruct((N, W16), jnp.float32),
            jax.ShapeDtypeStruct((N, W16), jnp.float32),
        ],
    )(cur, agg, wfes, wfen, bfe, gf, bf, gd, bd, m9, an, as_, bn, bs, c2n, c2s)


def _stage_b(agg, s2, wisn, wiss, bis16, e9):
    return pl.pallas_call(
        _stage_b_body,
        grid=(GRID,),
        in_specs=[
            _agg_spec(), _row_spec(W16),
            _full_spec((W16, W16)), _full_spec((W16, W16)),
            _full_spec((1, W16)), _full_spec((1, W16)),
        ],
        out_specs=[_row_spec(W16), _row_spec(W16)],
        out_shape=[
            jax.ShapeDtypeStruct((N, W16), jnp.float32),
            jax.ShapeDtypeStruct((N, W16), jnp.float32),
        ],
    )(agg, s2, wisn, wiss, bis16, e9)


def _stage_c(agg, s3, fpu, m9, e9):
    return pl.pallas_call(
        _stage_c_body,
        grid=(GRID,),
        in_specs=[
            _agg_spec(), _row_spec(W16), _row_spec(W16), _full_spec((1, W16)),
            _full_spec((1, W16)),
        ],
        out_specs=[_row_spec(D), _row_spec(W16)],
        out_shape=[
            jax.ShapeDtypeStruct((N, D), jnp.float32),
            jax.ShapeDtypeStruct((N, W16), jnp.float32),
        ],
    )(agg, s3, fpu, m9, e9)


def _pad16_vec(v):
    return jnp.pad(v, (0, W16 - v.shape[0])).reshape(1, W16)


def kernel(x, edge_index, pitch_speed, Wgp, bgp, Wg_f, bg_f, Wb_f, bb_f,
           Wg_d, bg_d, Wb_d, bb_d, Wfe_s, Wfe_n, bfe, Wds_s, Wds_n, bds,
           Wis_s, Wis_n, bis, num_timesteps):
    del num_timesteps
    fill = jnp.arange(EP - E, dtype=jnp.int32)
    src = jnp.concatenate([edge_index[0], fill % N]).reshape(NW * CPW, CH)
    dst = jnp.concatenate([edge_index[1],
                           N + fill % (NP - N)]).reshape(NW * CPW, CH)

    ge = jnp.tanh(pitch_speed @ Wgp + bgp)
    gf = (pitch_speed @ Wg_f + bg_f).reshape(1, H)
    bf = (pitch_speed @ Wb_f + bb_f).reshape(1, H)
    gd = _pad16_vec(pitch_speed @ Wg_d + bg_d)
    bd = _pad16_vec(pitch_speed @ Wb_d + bb_d)

    wfes = jnp.pad(Wfe_s, ((0, W16 - D), (0, 0)))
    wfen = jnp.pad(Wfe_n, ((0, W16 - D), (0, 0)))
    bfe_r = bfe.reshape(1, H)

    as_ = jnp.pad(Wds_s[:H], ((0, 0), (0, W16 - D)))
    an = jnp.pad(Wds_n[:H], ((0, 0), (0, W16 - D)))
    bs = jnp.pad(Wds_s[H:H + D], ((0, W16 - D), (0, W16 - D)))
    bn = jnp.pad(Wds_n[H:H + D], ((0, W16 - D), (0, W16 - D)))
    e9 = jnp.zeros((1, W16), jnp.float32).at[0, D].set(1.0)
    c2s = _pad16_vec(ge @ Wds_s[H + D:] + bds)
    c2n = _pad16_vec(ge @ Wds_n[H + D:]) + e9

    wiss = jnp.pad(Wis_s, ((0, W16 - D), (0, W16 - D)))
    wisn = jnp.pad(Wis_n, ((0, W16 - D), (0, W16 - D)))
    bis16 = _pad16_vec(bis)

    m9 = jnp.concatenate(
        [jnp.ones((1, D), jnp.float32),
         jnp.zeros((1, W16 - D), jnp.float32)], axis=1)

    zeros_rt = jnp.zeros((RPT, W16), jnp.float32)
    cur = jnp.concatenate(
        [x[:, :D], jnp.ones((N, 1), jnp.float32),
         jnp.zeros((N, W16 - D - 1), jnp.float32)], axis=1)

    preds = []
    for _t in range(4):
        agg1 = _sc_scatter(cur, src, dst, zeros_rt)
        p2, s2, fpu = _stage_a(cur, agg1, wfes, wfen, bfe_r, gf, bf, gd, bd,
                               m9, an, as_, bn, bs, c2n, c2s)
        agg2 = _sc_scatter(p2, src, dst, zeros_rt)
        p3, s3 = _stage_b(agg2, s2, wisn, wiss, bis16, e9)
        agg3 = _sc_scatter(p3, src, dst, zeros_rt)
        pred, cur = _stage_c(agg3, s3, fpu, m9, e9)
        preds.append(pred)

    return jnp.stack(preds)


# speedup vs baseline: 26.2441x; 1.0301x over previous
import functools

import jax
import jax.numpy as jnp
from jax import lax
from jax.experimental import pallas as pl
from jax.experimental.pallas import tpu as pltpu
from jax.experimental.pallas import tpu_sc as plsc

N = 100000
E = 1600000
D = 9
H = 128
G = 64
W16 = 16

NC = 2
NS = 16
NW = NC * NS
CH = 128
GB = 28
GPW = 14
CPW = GB * GPW
EP = NW * CPW * CH
NB = 7
NP = 100096
RPT = NP // NS

ZR = 136
BN = 5000
GRID = N // BN


@functools.cache
def _build_sc_scatter():
    mesh = plsc.VectorSubcoreMesh(core_axis_name="c", subcore_axis_name="s")

    @functools.partial(
        pl.kernel,
        out_type=jax.ShapeDtypeStruct((NC, NP, W16), jnp.float32),
        mesh=mesh,
        compiler_params=pltpu.CompilerParams(use_tc_tiling_on_sc=False),
        scratch_types=[
            pltpu.VMEM((2, GB, CH), jnp.int32),
            pltpu.VMEM((2, GB, CH), jnp.int32),
            pltpu.VMEM((NB, CH, W16), jnp.float32),
            pltpu.VMEM((ZR, W16), jnp.float32),
            pltpu.VMEM_SHARED((NP, W16), jnp.float32),
            pltpu.SemaphoreType.DMA((2,)),
            pltpu.SemaphoreType.DMA((NB,)),
            pltpu.SemaphoreType.DMA((NB,)),
        ],
    )
    def sc_scatter(payload_hbm, src_hbm, dst_hbm, out_hbm,
                   sidx, didx, rows, zbuf, acc, sem_i, sem_g, sem_s):
        c = lax.axis_index("c")
        s = lax.axis_index("s")
        w = c * NS + s
        row0 = s * RPT
        g0 = w * GPW

        def fire_group_fetch(g, gb):
            r = (g0 + g) * GB
            pltpu.async_copy(src_hbm.at[pl.ds(r, GB)], sidx.at[gb],
                             sem_i.at[gb])
            pltpu.async_copy(dst_hbm.at[pl.ds(r, GB)], didx.at[gb],
                             sem_i.at[gb])

        def wait_group_fetch(g, gb):
            r = (g0 + g) * GB
            pltpu.make_async_copy(src_hbm.at[pl.ds(r, GB)], sidx.at[gb],
                                  sem_i.at[gb]).wait()
            pltpu.make_async_copy(dst_hbm.at[pl.ds(r, GB)], didx.at[gb],
                                  sem_i.at[gb]).wait()

        fire_group_fetch(0, 0)

        def zero_zbuf(i, carry):
            zbuf[i, :] = jnp.zeros((W16,), jnp.float32)
            return carry

        lax.fori_loop(0, ZR, zero_zbuf, 0)

        def zero_acc(i, carry):
            pltpu.sync_copy(zbuf, acc.at[pl.ds(row0 + i * ZR, ZR)])
            return carry

        lax.fori_loop(0, RPT // ZR, zero_acc, 0)
        plsc.subcore_barrier()

        def group_body(g, carry):
            gb = lax.rem(g, 2)
            wait_group_fetch(g, gb)

            @pl.when(g + 1 < GPW)
            def _():
                fire_group_fetch(g + 1, lax.rem(g + 1, 2))

            for b in range(NB):
                pltpu.async_copy(payload_hbm.at[sidx.at[gb].at[b]],
                                 rows.at[b], sem_g.at[b])

            def chunk_body(j, carry2):
                b = lax.rem(j, NB)
                pltpu.make_async_copy(payload_hbm.at[sidx.at[gb].at[j]],
                                      rows.at[b], sem_g.at[b]).wait()
                pltpu.async_copy(rows.at[b], acc.at[didx.at[gb].at[j]],
                                 sem_s.at[b], add=True)
                pltpu.make_async_copy(rows.at[b], acc.at[didx.at[gb].at[j]],
                                      sem_s.at[b]).wait()

                @pl.when(j + NB < GB)
                def _():
                    pltpu.async_copy(payload_hbm.at[sidx.at[gb].at[j + NB]],
                                     rows.at[b], sem_g.at[b])

                return carry2

            lax.fori_loop(0, GB, chunk_body, 0)
            return carry

        lax.fori_loop(0, GPW, group_body, 0)

        plsc.subcore_barrier()
        pltpu.sync_copy(acc.at[pl.ds(row0, RPT)],
                        out_hbm.at[c, pl.ds(row0, RPT)])

    return sc_scatter


def _sc_scatter(payload, src, dst):
    return _build_sc_scatter()(payload, src, dst)


def _mean_from_partials(agg_ref):
    aggs = agg_ref[0] + agg_ref[1]
    deg = aggs[:, 9:10]
    inv = 1.0 / jnp.maximum(deg, 1.0)
    return aggs * inv


def _stage_a_body(cur_ref, agg_ref, wfes_ref, wfen_ref, bfe_ref,
                  gf_ref, bf_ref, gd_ref, bd_ref, m9_ref,
                  an_ref, as_ref, bn_ref, bs_ref, c2n_ref, c2s_ref,
                  p2_ref, s2_ref, fpu_ref):
    cur = cur_ref[...]
    mean1 = _mean_from_partials(agg_ref)
    pre = (jnp.dot(cur, wfes_ref[...], preferred_element_type=jnp.float32)
           + jnp.dot(mean1, wfen_ref[...], preferred_element_type=jnp.float32)
           + bfe_ref[...])
    learned = jnp.maximum(pre, 0.0)
    mu = jnp.mean(learned, axis=1, keepdims=True)
    var = jnp.mean((learned - mu) ** 2, axis=1, keepdims=True)
    ln = (learned - mu) * lax.rsqrt(var + 1e-5) * gf_ref[...] + bf_ref[...]

    m9 = m9_ref[...]
    mu9 = jnp.sum(cur * m9, axis=1, keepdims=True) / 9.0
    var9 = jnp.sum((cur - mu9) ** 2 * m9, axis=1, keepdims=True) / 9.0
    fpu = ((cur - mu9) * lax.rsqrt(var9 + 1e-5) * gd_ref[...] + bd_ref[...]) * m9

    p2_ref[...] = (jnp.dot(ln, an_ref[...], preferred_element_type=jnp.float32)
                   + jnp.dot(fpu, bn_ref[...], preferred_element_type=jnp.float32)
                   + c2n_ref[...])
    s2_ref[...] = (jnp.dot(ln, as_ref[...], preferred_element_type=jnp.float32)
                   + jnp.dot(fpu, bs_ref[...], preferred_element_type=jnp.float32)
                   + c2s_ref[...])
    fpu_ref[...] = fpu


def _stage_b_body(agg_ref, s2_ref, wisn_ref, wiss_ref, bis_ref, e9_ref,
                  p3_ref, s3_ref):
    fdot = s2_ref[...] + _mean_from_partials(agg_ref)
    p3_ref[...] = jnp.dot(fdot, wisn_ref[...],
                          preferred_element_type=jnp.float32) + e9_ref[...]
    s3_ref[...] = jnp.dot(fdot, wiss_ref[...],
                          preferred_element_type=jnp.float32) + bis_ref[...]


def _stage_c_body(agg_ref, s3_ref, fpu_ref, m9_ref, e9_ref, pred_ref, cur_ref):
    fint = s3_ref[...] + _mean_from_partials(agg_ref)
    fpred = (fpu_ref[...] + fint) * m9_ref[...]
    pred_ref[...] = fpred[:, :D]
    cur_ref[...] = fpred + e9_ref[...]


def _row_spec(width):
    return pl.BlockSpec((BN, width), lambda i: (i, 0))


def _agg_spec():
    return pl.BlockSpec((NC, BN, W16), lambda i: (0, i, 0))


def _full_spec(shape):
    nd = len(shape)
    return pl.BlockSpec(shape, lambda i, _n=nd: (0,) * _n)


def _stage_a(cur, agg, wfes, wfen, bfe, gf, bf, gd, bd, m9,
             an, as_, bn, bs, c2n, c2s):
    return pl.pallas_call(
        _stage_a_body,
        grid=(GRID,),
        in_specs=[
            _row_spec(W16), _agg_spec(),
            _full_spec((W16, H)), _full_spec((W16, H)), _full_spec((1, H)),
            _full_spec((1, H)), _full_spec((1, H)),
            _full_spec((1, W16)), _full_spec((1, W16)), _full_spec((1, W16)),
            _full_spec((H, W16)), _full_spec((H, W16)),
            _full_spec((W16, W16)), _full_spec((W16, W16)),
            _full_spec((1, W16)), _full_spec((1, W16)),
        ],
        out_specs=[_row_spec(W16), _row_spec(W16), _row_spec(W16)],
        out_shape=[
            jax.ShapeDtypeStruct((N, W16), jnp.float32),
            jax.ShapeDtypeStruct((N, W16), jnp.float32),
            jax.ShapeDtypeStruct((N, W16), jnp.float32),
        ],
    )(cur, agg, wfes, wfen, bfe, gf, bf, gd, bd, m9, an, as_, bn, bs, c2n, c2s)


def _stage_b(agg, s2, wisn, wiss, bis16, e9):
    return pl.pallas_call(
        _stage_b_body,
        grid=(GRID,),
        in_specs=[
            _agg_spec(), _row_spec(W16),
            _full_spec((W16, W16)), _full_spec((W16, W16)),
            _full_spec((1, W16)), _full_spec((1, W16)),
        ],
        out_specs=[_row_spec(W16), _row_spec(W16)],
        out_shape=[
            jax.ShapeDtypeStruct((N, W16), jnp.float32),
            jax.ShapeDtypeStruct((N, W16), jnp.float32),
        ],
    )(agg, s2, wisn, wiss, bis16, e9)


def _stage_c(agg, s3, fpu, m9, e9):
    return pl.pallas_call(
        _stage_c_body,
        grid=(GRID,),
        in_specs=[
            _agg_spec(), _row_spec(W16), _row_spec(W16), _full_spec((1, W16)),
            _full_spec((1, W16)),
        ],
        out_specs=[_row_spec(D), _row_spec(W16)],
        out_shape=[
            jax.ShapeDtypeStruct((N, D), jnp.float32),
            jax.ShapeDtypeStruct((N, W16), jnp.float32),
        ],
    )(agg, s3, fpu, m9, e9)


def _pad16_vec(v):
    return jnp.pad(v, (0, W16 - v.shape[0])).reshape(1, W16)


def kernel(x, edge_index, pitch_speed, Wgp, bgp, Wg_f, bg_f, Wb_f, bb_f,
           Wg_d, bg_d, Wb_d, bb_d, Wfe_s, Wfe_n, bfe, Wds_s, Wds_n, bds,
           Wis_s, Wis_n, bis, num_timesteps):
    del num_timesteps
    fill = jnp.arange(EP - E, dtype=jnp.int32)
    src = jnp.concatenate([edge_index[0], fill % N]).reshape(NW * CPW, CH)
    dst = jnp.concatenate([edge_index[1],
                           N + fill % (NP - N)]).reshape(NW * CPW, CH)

    ge = jnp.tanh(pitch_speed @ Wgp + bgp)
    gf = (pitch_speed @ Wg_f + bg_f).reshape(1, H)
    bf = (pitch_speed @ Wb_f + bb_f).reshape(1, H)
    gd = _pad16_vec(pitch_speed @ Wg_d + bg_d)
    bd = _pad16_vec(pitch_speed @ Wb_d + bb_d)

    wfes = jnp.pad(Wfe_s, ((0, W16 - D), (0, 0)))
    wfen = jnp.pad(Wfe_n, ((0, W16 - D), (0, 0)))
    bfe_r = bfe.reshape(1, H)

    as_ = jnp.pad(Wds_s[:H], ((0, 0), (0, W16 - D)))
    an = jnp.pad(Wds_n[:H], ((0, 0), (0, W16 - D)))
    bs = jnp.pad(Wds_s[H:H + D], ((0, W16 - D), (0, W16 - D)))
    bn = jnp.pad(Wds_n[H:H + D], ((0, W16 - D), (0, W16 - D)))
    e9 = jnp.zeros((1, W16), jnp.float32).at[0, D].set(1.0)
    c2s = _pad16_vec(ge @ Wds_s[H + D:] + bds)
    c2n = _pad16_vec(ge @ Wds_n[H + D:]) + e9

    wiss = jnp.pad(Wis_s, ((0, W16 - D), (0, W16 - D)))
    wisn = jnp.pad(Wis_n, ((0, W16 - D), (0, W16 - D)))
    bis16 = _pad16_vec(bis)

    m9 = jnp.concatenate(
        [jnp.ones((1, D), jnp.float32),
         jnp.zeros((1, W16 - D), jnp.float32)], axis=1)

    cur = jnp.concatenate(
        [x[:, :D], jnp.ones((N, 1), jnp.float32),
         jnp.zeros((N, W16 - D - 1), jnp.float32)], axis=1)

    preds = []
    for _t in range(4):
        agg1 = _sc_scatter(cur, src, dst)
        p2, s2, fpu = _stage_a(cur, agg1, wfes, wfen, bfe_r, gf, bf, gd, bd,
                               m9, an, as_, bn, bs, c2n, c2s)
        agg2 = _sc_scatter(p2, src, dst)
        p3, s3 = _stage_b(agg2, s2, wisn, wiss, bis16, e9)
        agg3 = _sc_scatter(p3, src, dst)
        pred, cur = _stage_c(agg3, s3, fpu, m9, e9)
        preds.append(pred)

    return jnp.stack(preds)
